# baseline (speedup 1.0000x reference)
import functools

import jax
import jax.numpy as jnp
from jax.experimental import pallas as pl
from jax.experimental.pallas import tpu as pltpu

_TILE_OUT = 2048
_NBUF = 4


def _linear_kernel(nsteps, last_rows, x_ref, w_hbm_ref, b_ref, o_ref, wbuf, sems):
    i = pl.program_id(0)

    def issue(step, static_kind=None):
        slot = jax.lax.rem(step, _NBUF)
        if static_kind == "full":
            pltpu.make_async_copy(
                w_hbm_ref.at[pl.ds(step * _TILE_OUT, _TILE_OUT)],
                wbuf.at[slot],
                sems.at[slot],
            ).start()
            return
        if static_kind == "partial":
            pltpu.make_async_copy(
                w_hbm_ref.at[pl.ds(step * _TILE_OUT, last_rows)],
                wbuf.at[slot, pl.ds(0, last_rows)],
                sems.at[slot],
            ).start()
            return

        @pl.when(step < nsteps - 1)
        def _():
            pltpu.make_async_copy(
                w_hbm_ref.at[pl.ds(step * _TILE_OUT, _TILE_OUT)],
                wbuf.at[slot],
                sems.at[slot],
            ).start()

        @pl.when(step == nsteps - 1)
        def _():
            pltpu.make_async_copy(
                w_hbm_ref.at[pl.ds(step * _TILE_OUT, last_rows)],
                wbuf.at[slot, pl.ds(0, last_rows)],
                sems.at[slot],
            ).start()

    @pl.when(i == 0)
    def _():
        for s in range(min(_NBUF, nsteps)):
            issue(s, static_kind="full" if s < nsteps - 1 else "partial")

    slot = jax.lax.rem(i, _NBUF)

    @pl.when(i < nsteps - 1)
    def _():
        pltpu.make_async_copy(
            w_hbm_ref.at[pl.ds(i * _TILE_OUT, _TILE_OUT)],
            wbuf.at[slot],
            sems.at[slot],
        ).wait()

    @pl.when(i == nsteps - 1)
    def _():
        pltpu.make_async_copy(
            w_hbm_ref.at[pl.ds(i * _TILE_OUT, last_rows)],
            wbuf.at[slot, pl.ds(0, last_rows)],
            sems.at[slot],
        ).wait()

    acc = jax.lax.dot_general(
        x_ref[...],
        wbuf[slot],
        dimension_numbers=(((1,), (1,)), ((), ())),
        preferred_element_type=jnp.float32,
    )
    o_ref[...] = acc + b_ref[...][None, :]

    issue(i + _NBUF)


@jax.jit
def kernel(x, W, b):
    batch, in_features = x.shape
    out_features = W.shape[0]
    grid = pl.cdiv(out_features, _TILE_OUT)
    last_rows = out_features - (grid - 1) * _TILE_OUT
    return pl.pallas_call(
        functools.partial(_linear_kernel, grid, last_rows),
        grid=(grid,),
        in_specs=[
            pl.BlockSpec((batch, in_features), lambda i: (0, 0)),
            pl.BlockSpec(memory_space=pltpu.MemorySpace.HBM),
            pl.BlockSpec((_TILE_OUT,), lambda i: (i,)),
        ],
        out_specs=pl.BlockSpec((batch, _TILE_OUT), lambda i: (0, i)),
        out_shape=jax.ShapeDtypeStruct((batch, out_features), jnp.float32),
        scratch_shapes=[
            pltpu.VMEM((_NBUF, _TILE_OUT, in_features), jnp.float32),
            pltpu.SemaphoreType.DMA((_NBUF,)),
        ],
        compiler_params=pltpu.CompilerParams(
            dimension_semantics=("arbitrary",),
        ),
    )(x, W, b)


# speedup vs baseline: 1.0125x; 1.0125x over previous
import functools

import jax
import jax.numpy as jnp
from jax.experimental import pallas as pl
from jax.experimental.pallas import tpu as pltpu

_TILE_OUT = 2048


def _linear_kernel(x_ref, w_ref, b_ref, o_ref):
    acc = jax.lax.dot_general(
        x_ref[...],
        w_ref[...],
        dimension_numbers=(((1,), (1,)), ((), ())),
        preferred_element_type=jnp.float32,
    )
    o_ref[...] = acc + b_ref[...][None, :]


@jax.jit
def kernel(x, W, b):
    batch, in_features = x.shape
    out_features = W.shape[0]
    grid = pl.cdiv(out_features, _TILE_OUT)
    return pl.pallas_call(
        _linear_kernel,
        grid=(grid,),
        in_specs=[
            pl.BlockSpec((batch, in_features), lambda i: (0, 0)),
            pl.BlockSpec((_TILE_OUT, in_features), lambda i: (i, 0)),
            pl.BlockSpec((_TILE_OUT,), lambda i: (i,)),
        ],
        out_specs=pl.BlockSpec((batch, _TILE_OUT), lambda i: (0, i)),
        out_shape=jax.ShapeDtypeStruct((batch, out_features), jnp.float32),
        compiler_params=pltpu.CompilerParams(
            dimension_semantics=("parallel",),
        ),
    )(x, W, b)


# speedup vs baseline: 1.0164x; 1.0039x over previous
import functools

import jax
import jax.numpy as jnp
from jax.experimental import pallas as pl
from jax.experimental.pallas import tpu as pltpu

_TILE_OUT = 2048


def _linear_kernel(x_ref, w_ref, b_ref, o_ref):
    acc = jax.lax.dot_general(
        x_ref[...],
        w_ref[...],
        dimension_numbers=(((1,), (1,)), ((), ())),
        preferred_element_type=jnp.float32,
    )
    o_ref[...] = acc + b_ref[...][None, :]


@jax.jit
def kernel(x, W, b):
    batch, in_features = x.shape
    out_features = W.shape[0]
    grid = pl.cdiv(out_features, _TILE_OUT)
    return pl.pallas_call(
        _linear_kernel,
        grid=(grid,),
        in_specs=[
            pl.BlockSpec((batch, in_features), lambda i: (0, 0)),
            pl.BlockSpec((_TILE_OUT, in_features), lambda i: (i, 0)),
            pl.BlockSpec((_TILE_OUT,), lambda i: (i,)),
        ],
        out_specs=pl.BlockSpec((batch, _TILE_OUT), lambda i: (0, i)),
        out_shape=jax.ShapeDtypeStruct((batch, out_features), jnp.float32),
        compiler_params=pltpu.CompilerParams(
            dimension_semantics=("arbitrary",),
        ),
    )(x, W, b)
